# baseline (speedup 1.0000x reference)
import functools

import jax
import jax.numpy as jnp
from jax import lax
from jax.experimental import pallas as pl
from jax.experimental.pallas import tpu as pltpu
from jax.experimental.pallas import tpu_sc as plsc

NUM_EXPERTS = 8
TOP_K = 2
D_MODEL = 1024
D_FF = 4096
NTOK = 4096
NE = NTOK * TOP_K
BLOCK = 256
NPAD = NE + NUM_EXPERTS * BLOCK
NB = NPAD // BLOCK
TOK_R, TOK_C = 32, 128

SC_NC = 2
SC_NW = 32
TOK_PER_W = NTOK // SC_NW
G_CHUNK = 32
C_CHUNK = 16


def _routing_kernel(idx0_ref, idx1_ref, ppos0_ref, ppos1_ref, binfo_ref):
    idx0 = idx0_ref[...]
    idx1 = idx1_ref[...]
    r = lax.broadcasted_iota(jnp.int32, (TOK_C, TOK_C), 0)
    c = lax.broadcasted_iota(jnp.int32, (TOK_C, TOK_C), 1)
    upper = (r < c).astype(jnp.float32)
    r2 = lax.broadcasted_iota(jnp.int32, (TOK_R, TOK_R), 0)
    c2 = lax.broadcasted_iota(jnp.int32, (TOK_R, TOK_R), 1)
    lower = (c2 < r2).astype(jnp.float32)
    ones = jnp.ones((TOK_C, TOK_C), jnp.float32)

    ppos0 = jnp.zeros((TOK_R, TOK_C), jnp.int32)
    ppos1 = jnp.zeros((TOK_R, TOK_C), jnp.int32)
    off = jnp.int32(0)
    ends = []
    for e in range(NUM_EXPERTS):
        m0 = (idx0 == e)
        m1 = (idx1 == e)
        m0f = m0.astype(jnp.float32)
        s = m0f + m1.astype(jnp.float32)
        within_row = jnp.dot(s, upper, preferred_element_type=jnp.float32)
        before_rows = jnp.dot(
            jnp.dot(lower, s, preferred_element_type=jnp.float32), ones,
            preferred_element_type=jnp.float32)
        pref = within_row + before_rows
        rank0 = pref.astype(jnp.int32)
        rank1 = (pref + m0f).astype(jnp.int32)
        cnt = jnp.sum(s).astype(jnp.int32)
        padded_cnt = ((cnt + BLOCK - 1) // BLOCK) * BLOCK
        ppos0 = ppos0 + m0.astype(jnp.int32) * (off + rank0)
        ppos1 = ppos1 + m1.astype(jnp.int32) * (off + rank1)
        off = off + padded_cnt
        ends.append(off)

    ppos0_ref[...] = ppos0
    ppos1_ref[...] = ppos1
    block_start = lax.broadcasted_iota(jnp.int32, (8, 128), 1) * BLOCK
    be = jnp.zeros((8, 128), jnp.int32)
    for e in range(NUM_EXPERTS):
        be = be + (block_start >= ends[e]).astype(jnp.int32)
    be = jnp.minimum(be, NUM_EXPERTS - 1)
    row = lax.broadcasted_iota(jnp.int32, (8, 128), 0)
    binfo_ref[...] = jnp.where(row == 0, be, off // BLOCK)


def _route(expert_indices):
    ei = expert_indices.astype(jnp.int32)
    idx0 = ei[:, 0].reshape(TOK_R, TOK_C)
    idx1 = ei[:, 1].reshape(TOK_R, TOK_C)
    return pl.pallas_call(
        _routing_kernel,
        out_shape=(
            jax.ShapeDtypeStruct((TOK_R, TOK_C), jnp.int32),
            jax.ShapeDtypeStruct((TOK_R, TOK_C), jnp.int32),
            jax.ShapeDtypeStruct((8, 128), jnp.int32),
        ),
    )(idx0, idx1)


def _gemm_kernel(be_ref, nact_ref, xg_ref, w1_ref, w2_ref, out_ref):
    del be_ref
    b = pl.program_id(0)
    f = pl.program_id(1)

    @pl.when(b < nact_ref[0])
    def _():
        mid = jax.nn.gelu(
            jnp.dot(xg_ref[...].astype(jnp.bfloat16), w1_ref[0],
                    preferred_element_type=jnp.float32))
        part = jnp.dot(mid.astype(jnp.bfloat16), w2_ref[0],
                       preferred_element_type=jnp.float32)

        @pl.when(f == 0)
        def _():
            out_ref[...] = part

        @pl.when(f != 0)
        def _():
            out_ref[...] = out_ref[...] + part


def _grouped_mlp(block_expert, nact, gathered, w1b, w2b):
    grid_spec = pltpu.PrefetchScalarGridSpec(
        num_scalar_prefetch=2,
        grid=(NB, 2),
        in_specs=[
            pl.BlockSpec((BLOCK, D_MODEL), lambda b, f, be, na: (b, 0)),
            pl.BlockSpec((1, D_MODEL, D_FF // 2),
                         lambda b, f, be, na: (be[b], 0, f)),
            pl.BlockSpec((1, D_FF // 2, D_MODEL),
                         lambda b, f, be, na: (be[b], f, 0)),
        ],
        out_specs=pl.BlockSpec((BLOCK, D_MODEL), lambda b, f, be, na: (b, 0)),
    )
    return pl.pallas_call(
        _gemm_kernel,
        grid_spec=grid_spec,
        out_shape=jax.ShapeDtypeStruct((NPAD, D_MODEL), jnp.float32),
    )(block_expert, nact, gathered, w1b, w2b)


def _sc_gather(x_flat, ppos0_2d, ppos1_2d):
    mesh = plsc.VectorSubcoreMesh(core_axis_name="c", subcore_axis_name="s")
    n_ch = TOK_PER_W // G_CHUNK
    n_buf = 3

    @functools.partial(
        pl.kernel, mesh=mesh,
        out_type=jax.ShapeDtypeStruct((NPAD, D_MODEL), jnp.float32),
        scratch_types=[
            pltpu.VMEM((n_ch, G_CHUNK), jnp.int32),
            pltpu.VMEM((n_ch, G_CHUNK), jnp.int32),
        ] + [pltpu.VMEM((G_CHUNK, D_MODEL), jnp.float32)] * n_buf + [
            pltpu.SemaphoreType.DMA,
            pltpu.SemaphoreType.DMA,
            pltpu.SemaphoreType.DMA,
        ],
    )
    def k(x_hbm, p0_hbm, p1_hbm, out_hbm, p0_v, p1_v,
          buf0, buf1, buf2, semg, sem0, sem1):
        bufs = [buf0, buf1, buf2]
        wid = lax.axis_index("s") * SC_NC + lax.axis_index("c")
        base = wid * TOK_PER_W
        pltpu.sync_copy(p0_hbm.at[pl.ds(wid * n_ch, n_ch)], p0_v)
        pltpu.sync_copy(p1_hbm.at[pl.ds(wid * n_ch, n_ch)], p1_v)

        def gstart(c):
            return pltpu.async_copy(
                x_hbm.at[pl.ds(base + c * G_CHUNK, G_CHUNK)],
                bufs[c % n_buf], semg)

        g, sc0, sc1 = {}, {}, {}
        g[0] = gstart(0)
        g[1] = gstart(1)
        for c in range(n_ch):
            g[c].wait()
            n = c + 2
            if n < n_ch:
                if n - n_buf >= 0:
                    sc0[n - n_buf].wait()
                    sc1[n - n_buf].wait()
                g[n] = gstart(n)
            sc0[c] = pltpu.async_copy(
                bufs[c % n_buf], out_hbm.at[p0_v.at[c]], sem0)
            sc1[c] = pltpu.async_copy(
                bufs[c % n_buf], out_hbm.at[p1_v.at[c]], sem1)
        for c in range(max(0, n_ch - n_buf), n_ch):
            sc0[c].wait()
            sc1[c].wait()

    return k(x_flat, ppos0_2d, ppos1_2d)


def _sc_combine(out_perm, ppos0, ppos1, ew0, ew1):
    mesh = plsc.VectorSubcoreMesh(core_axis_name="c", subcore_axis_name="s")
    n_ch = TOK_PER_W // C_CHUNK
    n_grp = D_MODEL // 64

    @functools.partial(
        pl.kernel, mesh=mesh,
        out_type=jax.ShapeDtypeStruct((NTOK, D_MODEL), jnp.float32),
        scratch_types=[
            pltpu.VMEM((TOK_PER_W,), jnp.int32),
            pltpu.VMEM((TOK_PER_W,), jnp.int32),
            pltpu.VMEM((TOK_PER_W, 16), jnp.float32),
            pltpu.VMEM((TOK_PER_W, 16), jnp.float32),
            pltpu.VMEM((C_CHUNK, D_MODEL), jnp.float32),
            pltpu.VMEM((C_CHUNK, D_MODEL), jnp.float32),
            pltpu.VMEM((C_CHUNK, D_MODEL), jnp.float32),
            pltpu.VMEM((C_CHUNK, D_MODEL), jnp.float32),
            pltpu.VMEM((C_CHUNK, D_MODEL), jnp.float32),
            pltpu.SemaphoreType.DMA,
            pltpu.SemaphoreType.DMA,
        ],
    )
    def k(op_hbm, p0_hbm, p1_hbm, w0_hbm, w1_hbm, res_hbm,
          p0_v, p1_v, w0_v, w1_v,
          r0_a, r1_a, r0_b, r1_b, res_a,
          sem_a, sem_b):
        wid = lax.axis_index("s") * SC_NC + lax.axis_index("c")
        base = wid * TOK_PER_W
        pltpu.sync_copy(p0_hbm.at[pl.ds(base, TOK_PER_W)], p0_v)
        pltpu.sync_copy(p1_hbm.at[pl.ds(base, TOK_PER_W)], p1_v)
        pltpu.sync_copy(w0_hbm.at[pl.ds(base, TOK_PER_W)], w0_v)
        pltpu.sync_copy(w1_hbm.at[pl.ds(base, TOK_PER_W)], w1_v)

        def start(c):
            r0, r1 = (r0_a, r1_a) if c % 2 == 0 else (r0_b, r1_b)
            sem = sem_a if c % 2 == 0 else sem_b
            g0 = pltpu.async_copy(
                op_hbm.at[p0_v.at[pl.ds(c * C_CHUNK, C_CHUNK)]], r0, sem)
            g1 = pltpu.async_copy(
                op_hbm.at[p1_v.at[pl.ds(c * C_CHUNK, C_CHUNK)]], r1, sem)
            return g0, g1

        pending = start(0)
        for c in range(n_ch):
            r0, r1 = (r0_a, r1_a) if c % 2 == 0 else (r0_b, r1_b)
            res = res_a
            g0, g1 = pending
            if c + 1 < n_ch:
                pending = start(c + 1)
            g0.wait()
            g1.wait()

            def token_body(t, _):
                w0b = w0_v[c * C_CHUNK + t, pl.ds(0, 16)]
                w1b = w1_v[c * C_CHUNK + t, pl.ds(0, 16)]

                def vec_body(v, _):
                    for u in range(4):
                        sl = pl.ds(v * 64 + u * 16, 16)
                        res[t, sl] = w0b * r0[t, sl] + w1b * r1[t, sl]
                    return 0

                lax.fori_loop(0, n_grp, vec_body, 0)
                return 0

            lax.fori_loop(0, C_CHUNK, token_body, 0)
            pltpu.sync_copy(
                res, res_hbm.at[pl.ds(base + c * C_CHUNK, C_CHUNK)])

    return k(out_perm, ppos0, ppos1, ew0, ew1)


def kernel(x, expert_weights, expert_indices, w1, w2):
    in_shape = x.shape
    x_flat = x.reshape(NTOK, D_MODEL)

    w1b = w1.astype(jnp.bfloat16)
    w2b = w2.astype(jnp.bfloat16)

    ppos0_2d, ppos1_2d, binfo = _route(expert_indices)
    block_expert = binfo[0, :NB]
    nact = binfo[1, :1]

    gathered = _sc_gather(
        x_flat,
        ppos0_2d.reshape(NTOK // G_CHUNK, G_CHUNK),
        ppos1_2d.reshape(NTOK // G_CHUNK, G_CHUNK))
    out_perm = _grouped_mlp(block_expert, nact, gathered, w1b, w2b)

    ew = expert_weights.astype(jnp.float32)
    ew0 = jnp.broadcast_to(ew[:, 0:1], (NTOK, 16))
    ew1 = jnp.broadcast_to(ew[:, 1:2], (NTOK, 16))
    result = _sc_combine(
        out_perm, ppos0_2d.reshape(NTOK), ppos1_2d.reshape(NTOK), ew0, ew1)
    return result.reshape(in_shape)


# speedup vs baseline: 1.1513x; 1.1513x over previous
import functools

import jax
import jax.numpy as jnp
from jax import lax
from jax.experimental import pallas as pl
from jax.experimental.pallas import tpu as pltpu
from jax.experimental.pallas import tpu_sc as plsc

NUM_EXPERTS = 8
TOP_K = 2
D_MODEL = 1024
D_FF = 4096
NTOK = 4096
NE = NTOK * TOP_K
BLOCK = 256
NPAD = NE + NUM_EXPERTS * BLOCK
NB = NPAD // BLOCK
TOK_R, TOK_C = 32, 128

SC_NC = 2
SC_NW = 32
TOK_PER_W = NTOK // SC_NW
G_CHUNK = 32
C_CHUNK = 16


def _routing_kernel(idx0_ref, idx1_ref, ppos0_ref, ppos1_ref, binfo_ref):
    idx0 = idx0_ref[...]
    idx1 = idx1_ref[...]
    r = lax.broadcasted_iota(jnp.int32, (TOK_C, TOK_C), 0)
    c = lax.broadcasted_iota(jnp.int32, (TOK_C, TOK_C), 1)
    upper = (r < c).astype(jnp.float32)
    r2 = lax.broadcasted_iota(jnp.int32, (TOK_R, TOK_R), 0)
    c2 = lax.broadcasted_iota(jnp.int32, (TOK_R, TOK_R), 1)
    lower = (c2 < r2).astype(jnp.float32)
    ones = jnp.ones((TOK_C, TOK_C), jnp.float32)

    ppos0 = jnp.zeros((TOK_R, TOK_C), jnp.int32)
    ppos1 = jnp.zeros((TOK_R, TOK_C), jnp.int32)
    off = jnp.int32(0)
    ends = []
    for e in range(NUM_EXPERTS):
        m0 = (idx0 == e)
        m1 = (idx1 == e)
        m0f = m0.astype(jnp.float32)
        s = m0f + m1.astype(jnp.float32)
        within_row = jnp.dot(s, upper, preferred_element_type=jnp.float32)
        before_rows = jnp.dot(
            jnp.dot(lower, s, preferred_element_type=jnp.float32), ones,
            preferred_element_type=jnp.float32)
        pref = within_row + before_rows
        rank0 = pref.astype(jnp.int32)
        rank1 = (pref + m0f).astype(jnp.int32)
        cnt = jnp.sum(s).astype(jnp.int32)
        padded_cnt = ((cnt + BLOCK - 1) // BLOCK) * BLOCK
        ppos0 = ppos0 + m0.astype(jnp.int32) * (off + rank0)
        ppos1 = ppos1 + m1.astype(jnp.int32) * (off + rank1)
        off = off + padded_cnt
        ends.append(off)

    ppos0_ref[...] = ppos0
    ppos1_ref[...] = ppos1
    block_start = lax.broadcasted_iota(jnp.int32, (8, 128), 1) * BLOCK
    be = jnp.zeros((8, 128), jnp.int32)
    for e in range(NUM_EXPERTS):
        be = be + (block_start >= ends[e]).astype(jnp.int32)
    be = jnp.minimum(be, NUM_EXPERTS - 1)
    row = lax.broadcasted_iota(jnp.int32, (8, 128), 0)
    binfo_ref[...] = jnp.where(row == 0, be, off // BLOCK)


def _route(expert_indices):
    ei = expert_indices.astype(jnp.int32)
    idx0 = ei[:, 0].reshape(TOK_R, TOK_C)
    idx1 = ei[:, 1].reshape(TOK_R, TOK_C)
    return pl.pallas_call(
        _routing_kernel,
        out_shape=(
            jax.ShapeDtypeStruct((TOK_R, TOK_C), jnp.int32),
            jax.ShapeDtypeStruct((TOK_R, TOK_C), jnp.int32),
            jax.ShapeDtypeStruct((8, 128), jnp.int32),
        ),
    )(idx0, idx1)


def _gemm_kernel(be_ref, nact_ref, xg_ref, w1_ref, w2_ref, out_ref):
    del be_ref
    b = pl.program_id(0)

    @pl.when(b < nact_ref[0])
    def _():
        mid = jax.nn.gelu(
            jnp.dot(xg_ref[...].astype(jnp.bfloat16), w1_ref[0],
                    preferred_element_type=jnp.float32))
        out_ref[...] = jnp.dot(mid.astype(jnp.bfloat16), w2_ref[0],
                               preferred_element_type=jnp.float32)


def _grouped_mlp(block_expert, nact, gathered, w1b, w2b):
    grid_spec = pltpu.PrefetchScalarGridSpec(
        num_scalar_prefetch=2,
        grid=(NB,),
        in_specs=[
            pl.BlockSpec((BLOCK, D_MODEL), lambda b, be, na: (b, 0)),
            pl.BlockSpec((1, D_MODEL, D_FF), lambda b, be, na: (be[b], 0, 0)),
            pl.BlockSpec((1, D_FF, D_MODEL), lambda b, be, na: (be[b], 0, 0)),
        ],
        out_specs=pl.BlockSpec((BLOCK, D_MODEL), lambda b, be, na: (b, 0)),
    )
    return pl.pallas_call(
        _gemm_kernel,
        grid_spec=grid_spec,
        out_shape=jax.ShapeDtypeStruct((NPAD, D_MODEL), jnp.float32),
    )(block_expert, nact, gathered, w1b, w2b)


def _cast_kernel(w1_ref, w2_ref, o1_ref, o2_ref):
    o1_ref[...] = w1_ref[...].astype(jnp.bfloat16)
    o2_ref[...] = w2_ref[...].astype(jnp.bfloat16)


def _cast_weights(w1, w2):
    return pl.pallas_call(
        _cast_kernel,
        grid=(NUM_EXPERTS, 4),
        in_specs=[
            pl.BlockSpec((1, D_MODEL // 4, D_FF), lambda e, i: (e, i, 0)),
            pl.BlockSpec((1, D_FF // 4, D_MODEL), lambda e, i: (e, i, 0)),
        ],
        out_specs=[
            pl.BlockSpec((1, D_MODEL // 4, D_FF), lambda e, i: (e, i, 0)),
            pl.BlockSpec((1, D_FF // 4, D_MODEL), lambda e, i: (e, i, 0)),
        ],
        out_shape=[
            jax.ShapeDtypeStruct((NUM_EXPERTS, D_MODEL, D_FF), jnp.bfloat16),
            jax.ShapeDtypeStruct((NUM_EXPERTS, D_FF, D_MODEL), jnp.bfloat16),
        ],
    )(w1, w2)


def _sc_gather(x_flat, ppos0_2d, ppos1_2d):
    mesh = plsc.VectorSubcoreMesh(core_axis_name="c", subcore_axis_name="s")
    n_ch = TOK_PER_W // G_CHUNK
    n_buf = 3

    @functools.partial(
        pl.kernel, mesh=mesh,
        out_type=jax.ShapeDtypeStruct((NPAD, D_MODEL), jnp.float32),
        scratch_types=[
            pltpu.VMEM((n_ch, G_CHUNK), jnp.int32),
            pltpu.VMEM((n_ch, G_CHUNK), jnp.int32),
        ] + [pltpu.VMEM((G_CHUNK, D_MODEL), jnp.float32)] * n_buf + [
            pltpu.SemaphoreType.DMA,
            pltpu.SemaphoreType.DMA,
            pltpu.SemaphoreType.DMA,
        ],
    )
    def k(x_hbm, p0_hbm, p1_hbm, out_hbm, p0_v, p1_v,
          buf0, buf1, buf2, semg, sem0, sem1):
        bufs = [buf0, buf1, buf2]
        wid = lax.axis_index("s") * SC_NC + lax.axis_index("c")
        base = wid * TOK_PER_W
        pltpu.sync_copy(p0_hbm.at[pl.ds(wid * n_ch, n_ch)], p0_v)
        pltpu.sync_copy(p1_hbm.at[pl.ds(wid * n_ch, n_ch)], p1_v)

        def gstart(c):
            return pltpu.async_copy(
                x_hbm.at[pl.ds(base + c * G_CHUNK, G_CHUNK)],
                bufs[c % n_buf], semg)

        g, sc0, sc1 = {}, {}, {}
        g[0] = gstart(0)
        g[1] = gstart(1)
        for c in range(n_ch):
            g[c].wait()
            n = c + 2
            if n < n_ch:
                if n - n_buf >= 0:
                    sc0[n - n_buf].wait()
                    sc1[n - n_buf].wait()
                g[n] = gstart(n)
            sc0[c] = pltpu.async_copy(
                bufs[c % n_buf], out_hbm.at[p0_v.at[c]], sem0)
            sc1[c] = pltpu.async_copy(
                bufs[c % n_buf], out_hbm.at[p1_v.at[c]], sem1)
        for c in range(max(0, n_ch - n_buf), n_ch):
            sc0[c].wait()
            sc1[c].wait()

    return k(x_flat, ppos0_2d, ppos1_2d)


def _sc_combine(out_perm, ppos0, ppos1, ew0, ew1):
    mesh = plsc.VectorSubcoreMesh(core_axis_name="c", subcore_axis_name="s")
    n_ch = TOK_PER_W // C_CHUNK
    n_grp = D_MODEL // 64

    @functools.partial(
        pl.kernel, mesh=mesh,
        out_type=jax.ShapeDtypeStruct((NTOK, D_MODEL), jnp.float32),
        scratch_types=[
            pltpu.VMEM((TOK_PER_W,), jnp.int32),
            pltpu.VMEM((TOK_PER_W,), jnp.int32),
            pltpu.VMEM((TOK_PER_W, 16), jnp.float32),
            pltpu.VMEM((TOK_PER_W, 16), jnp.float32),
            pltpu.VMEM((C_CHUNK, D_MODEL), jnp.float32),
            pltpu.VMEM((C_CHUNK, D_MODEL), jnp.float32),
            pltpu.VMEM((C_CHUNK, D_MODEL), jnp.float32),
            pltpu.VMEM((C_CHUNK, D_MODEL), jnp.float32),
            pltpu.VMEM((C_CHUNK, D_MODEL), jnp.float32),
            pltpu.SemaphoreType.DMA,
            pltpu.SemaphoreType.DMA,
        ],
    )
    def k(op_hbm, p0_hbm, p1_hbm, w0_hbm, w1_hbm, res_hbm,
          p0_v, p1_v, w0_v, w1_v,
          r0_a, r1_a, r0_b, r1_b, res_a,
          sem_a, sem_b):
        wid = lax.axis_index("s") * SC_NC + lax.axis_index("c")
        base = wid * TOK_PER_W
        pltpu.sync_copy(p0_hbm.at[pl.ds(base, TOK_PER_W)], p0_v)
        pltpu.sync_copy(p1_hbm.at[pl.ds(base, TOK_PER_W)], p1_v)
        pltpu.sync_copy(w0_hbm.at[pl.ds(base, TOK_PER_W)], w0_v)
        pltpu.sync_copy(w1_hbm.at[pl.ds(base, TOK_PER_W)], w1_v)

        def start(c):
            r0, r1 = (r0_a, r1_a) if c % 2 == 0 else (r0_b, r1_b)
            sem = sem_a if c % 2 == 0 else sem_b
            g0 = pltpu.async_copy(
                op_hbm.at[p0_v.at[pl.ds(c * C_CHUNK, C_CHUNK)]], r0, sem)
            g1 = pltpu.async_copy(
                op_hbm.at[p1_v.at[pl.ds(c * C_CHUNK, C_CHUNK)]], r1, sem)
            return g0, g1

        pending = start(0)
        for c in range(n_ch):
            r0, r1 = (r0_a, r1_a) if c % 2 == 0 else (r0_b, r1_b)
            res = res_a
            g0, g1 = pending
            if c + 1 < n_ch:
                pending = start(c + 1)
            g0.wait()
            g1.wait()

            def token_body(t, _):
                w0b = w0_v[c * C_CHUNK + t, pl.ds(0, 16)]
                w1b = w1_v[c * C_CHUNK + t, pl.ds(0, 16)]

                def vec_body(v, _):
                    for u in range(4):
                        sl = pl.ds(v * 64 + u * 16, 16)
                        res[t, sl] = w0b * r0[t, sl] + w1b * r1[t, sl]
                    return 0

                lax.fori_loop(0, n_grp, vec_body, 0)
                return 0

            lax.fori_loop(0, C_CHUNK, token_body, 0)
            pltpu.sync_copy(
                res, res_hbm.at[pl.ds(base + c * C_CHUNK, C_CHUNK)])

    return k(out_perm, ppos0, ppos1, ew0, ew1)


def kernel(x, expert_weights, expert_indices, w1, w2):
    in_shape = x.shape
    x_flat = x.reshape(NTOK, D_MODEL)

    w1b, w2b = _cast_weights(w1, w2)

    ppos0_2d, ppos1_2d, binfo = _route(expert_indices)
    block_expert = binfo[0, :NB]
    nact = binfo[1, :1]

    gathered = _sc_gather(
        x_flat,
        ppos0_2d.reshape(NTOK // G_CHUNK, G_CHUNK),
        ppos1_2d.reshape(NTOK // G_CHUNK, G_CHUNK))
    out_perm = _grouped_mlp(block_expert, nact, gathered, w1b, w2b)

    ew = expert_weights.astype(jnp.float32)
    ew0 = jnp.broadcast_to(ew[:, 0:1], (NTOK, 16))
    ew1 = jnp.broadcast_to(ew[:, 1:2], (NTOK, 16))
    result = _sc_combine(
        out_perm, ppos0_2d.reshape(NTOK), ppos1_2d.reshape(NTOK), ew0, ew1)
    return result.reshape(in_shape)


# speedup vs baseline: 1.2029x; 1.0448x over previous
import functools

import jax
import jax.numpy as jnp
from jax import lax
from jax.experimental import pallas as pl
from jax.experimental.pallas import tpu as pltpu
from jax.experimental.pallas import tpu_sc as plsc

NUM_EXPERTS = 8
TOP_K = 2
D_MODEL = 1024
D_FF = 4096
NTOK = 4096
NE = NTOK * TOP_K
BLOCK = 256
NPAD = NE + NUM_EXPERTS * BLOCK
NB = NPAD // BLOCK
TOK_R, TOK_C = 32, 128

SC_NC = 2
SC_NW = 32
TOK_PER_W = NTOK // SC_NW
G_CHUNK = 32
C_CHUNK = 16


def _routing_kernel(idx0_ref, idx1_ref, ppos0_ref, ppos1_ref, binfo_ref):
    idx0 = idx0_ref[...]
    idx1 = idx1_ref[...]
    r = lax.broadcasted_iota(jnp.int32, (TOK_C, TOK_C), 0)
    c = lax.broadcasted_iota(jnp.int32, (TOK_C, TOK_C), 1)
    upper = (r < c).astype(jnp.float32)
    r2 = lax.broadcasted_iota(jnp.int32, (TOK_R, TOK_R), 0)
    c2 = lax.broadcasted_iota(jnp.int32, (TOK_R, TOK_R), 1)
    lower = (c2 < r2).astype(jnp.float32)
    ones = jnp.ones((TOK_C, TOK_C), jnp.float32)

    ppos0 = jnp.zeros((TOK_R, TOK_C), jnp.int32)
    ppos1 = jnp.zeros((TOK_R, TOK_C), jnp.int32)
    off = jnp.int32(0)
    ends = []
    for e in range(NUM_EXPERTS):
        m0 = (idx0 == e)
        m1 = (idx1 == e)
        m0f = m0.astype(jnp.float32)
        s = m0f + m1.astype(jnp.float32)
        within_row = jnp.dot(s, upper, preferred_element_type=jnp.float32)
        before_rows = jnp.dot(
            jnp.dot(lower, s, preferred_element_type=jnp.float32), ones,
            preferred_element_type=jnp.float32)
        pref = within_row + before_rows
        rank0 = pref.astype(jnp.int32)
        rank1 = (pref + m0f).astype(jnp.int32)
        cnt = jnp.sum(s).astype(jnp.int32)
        padded_cnt = ((cnt + BLOCK - 1) // BLOCK) * BLOCK
        ppos0 = ppos0 + m0.astype(jnp.int32) * (off + rank0)
        ppos1 = ppos1 + m1.astype(jnp.int32) * (off + rank1)
        off = off + padded_cnt
        ends.append(off)

    ppos0_ref[...] = ppos0
    ppos1_ref[...] = ppos1
    block_start = lax.broadcasted_iota(jnp.int32, (8, 128), 1) * BLOCK
    be = jnp.zeros((8, 128), jnp.int32)
    for e in range(NUM_EXPERTS):
        be = be + (block_start >= ends[e]).astype(jnp.int32)
    be = jnp.minimum(be, NUM_EXPERTS - 1)
    row = lax.broadcasted_iota(jnp.int32, (8, 128), 0)
    binfo_ref[...] = jnp.where(row == 0, be, off // BLOCK)


def _route(expert_indices):
    ei = expert_indices.astype(jnp.int32)
    idx0 = ei[:, 0].reshape(TOK_R, TOK_C)
    idx1 = ei[:, 1].reshape(TOK_R, TOK_C)
    return pl.pallas_call(
        _routing_kernel,
        out_shape=(
            jax.ShapeDtypeStruct((TOK_R, TOK_C), jnp.int32),
            jax.ShapeDtypeStruct((TOK_R, TOK_C), jnp.int32),
            jax.ShapeDtypeStruct((8, 128), jnp.int32),
        ),
    )(idx0, idx1)


def _gemm_kernel(be_ref, nact_ref, xg_ref, w1_ref, w2_ref, out_ref):
    del be_ref
    b = pl.program_id(0)

    @pl.when(b < nact_ref[0])
    def _():
        mid = jax.nn.gelu(
            jnp.dot(xg_ref[...].astype(jnp.bfloat16), w1_ref[0],
                    preferred_element_type=jnp.float32))
        out_ref[...] = jnp.dot(mid.astype(jnp.bfloat16), w2_ref[0],
                               preferred_element_type=jnp.float32)


def _grouped_mlp(block_expert, nact, gathered, w1b, w2b):
    grid_spec = pltpu.PrefetchScalarGridSpec(
        num_scalar_prefetch=2,
        grid=(NB,),
        in_specs=[
            pl.BlockSpec((BLOCK, D_MODEL), lambda b, be, na: (b, 0)),
            pl.BlockSpec((1, D_MODEL, D_FF), lambda b, be, na: (be[b], 0, 0)),
            pl.BlockSpec((1, D_FF, D_MODEL), lambda b, be, na: (be[b], 0, 0)),
        ],
        out_specs=pl.BlockSpec((BLOCK, D_MODEL), lambda b, be, na: (b, 0)),
    )
    return pl.pallas_call(
        _gemm_kernel,
        grid_spec=grid_spec,
        out_shape=jax.ShapeDtypeStruct((NPAD, D_MODEL), jnp.float32),
    )(block_expert, nact, gathered, w1b, w2b)


def _cast_kernel(w1_ref, w2_ref, o1_ref, o2_ref):
    o1_ref[...] = w1_ref[...].astype(jnp.bfloat16)
    o2_ref[...] = w2_ref[...].astype(jnp.bfloat16)


def _cast_weights(w1, w2):
    return pl.pallas_call(
        _cast_kernel,
        grid=(NUM_EXPERTS, 4),
        in_specs=[
            pl.BlockSpec((1, D_MODEL // 4, D_FF), lambda e, i: (e, i, 0)),
            pl.BlockSpec((1, D_FF // 4, D_MODEL), lambda e, i: (e, i, 0)),
        ],
        out_specs=[
            pl.BlockSpec((1, D_MODEL // 4, D_FF), lambda e, i: (e, i, 0)),
            pl.BlockSpec((1, D_FF // 4, D_MODEL), lambda e, i: (e, i, 0)),
        ],
        out_shape=[
            jax.ShapeDtypeStruct((NUM_EXPERTS, D_MODEL, D_FF), jnp.bfloat16),
            jax.ShapeDtypeStruct((NUM_EXPERTS, D_FF, D_MODEL), jnp.bfloat16),
        ],
    )(w1, w2)


def _sc_gather(x_flat, ppos0_2d, ppos1_2d):
    mesh = plsc.VectorSubcoreMesh(core_axis_name="c", subcore_axis_name="s")
    n_ch = TOK_PER_W // G_CHUNK
    n_buf = 3

    @functools.partial(
        pl.kernel, mesh=mesh,
        out_type=jax.ShapeDtypeStruct((NPAD, D_MODEL), jnp.float32),
        scratch_types=[
            pltpu.VMEM((n_ch, G_CHUNK), jnp.int32),
            pltpu.VMEM((n_ch, G_CHUNK), jnp.int32),
        ] + [pltpu.VMEM((G_CHUNK, D_MODEL), jnp.float32)] * n_buf + [
            pltpu.SemaphoreType.DMA,
            pltpu.SemaphoreType.DMA,
            pltpu.SemaphoreType.DMA,
        ],
    )
    def k(x_hbm, p0_hbm, p1_hbm, out_hbm, p0_v, p1_v,
          buf0, buf1, buf2, semg, sem0, sem1):
        bufs = [buf0, buf1, buf2]
        wid = lax.axis_index("s") * SC_NC + lax.axis_index("c")
        base = wid * TOK_PER_W
        pltpu.sync_copy(p0_hbm.at[pl.ds(wid * n_ch, n_ch)], p0_v)
        pltpu.sync_copy(p1_hbm.at[pl.ds(wid * n_ch, n_ch)], p1_v)

        half = G_CHUNK // 2

        def gstart(c):
            s0 = (base + c * G_CHUNK) // 2
            buf = bufs[c % n_buf]
            a = pltpu.async_copy(
                x_hbm.at[pl.ds(s0, half), 0], buf.at[pl.ds(0, half)], semg)
            b = pltpu.async_copy(
                x_hbm.at[pl.ds(s0, half), 1], buf.at[pl.ds(half, half)],
                semg)
            return (a, b)

        g, sc0, sc1 = {}, {}, {}
        g[0] = gstart(0)
        g[1] = gstart(1)
        for c in range(n_ch):
            g[c][0].wait()
            g[c][1].wait()
            n = c + 2
            if n < n_ch:
                if n - n_buf >= 0:
                    sc0[n - n_buf].wait()
                    sc1[n - n_buf].wait()
                g[n] = gstart(n)
            sc0[c] = pltpu.async_copy(
                bufs[c % n_buf], out_hbm.at[p0_v.at[c]], sem0)
            sc1[c] = pltpu.async_copy(
                bufs[c % n_buf], out_hbm.at[p1_v.at[c]], sem1)
        for c in range(max(0, n_ch - n_buf), n_ch):
            sc0[c].wait()
            sc1[c].wait()

    return k(x_flat, ppos0_2d, ppos1_2d)


def _sc_combine(out_perm, ppos0, ppos1, ew0, ew1):
    mesh = plsc.VectorSubcoreMesh(core_axis_name="c", subcore_axis_name="s")
    n_ch = TOK_PER_W // C_CHUNK
    n_grp = D_MODEL // 64

    @functools.partial(
        pl.kernel, mesh=mesh,
        out_type=jax.ShapeDtypeStruct((NTOK, D_MODEL), jnp.float32),
        scratch_types=[
            pltpu.VMEM((TOK_PER_W,), jnp.int32),
            pltpu.VMEM((TOK_PER_W,), jnp.int32),
            pltpu.VMEM((TOK_PER_W, 16), jnp.float32),
            pltpu.VMEM((TOK_PER_W, 16), jnp.float32),
            pltpu.VMEM((C_CHUNK, D_MODEL), jnp.float32),
            pltpu.VMEM((C_CHUNK, D_MODEL), jnp.float32),
            pltpu.VMEM((C_CHUNK, D_MODEL), jnp.float32),
            pltpu.VMEM((C_CHUNK, D_MODEL), jnp.float32),
            pltpu.VMEM((C_CHUNK, D_MODEL), jnp.float32),
            pltpu.SemaphoreType.DMA,
            pltpu.SemaphoreType.DMA,
        ],
    )
    def k(op_hbm, p0_hbm, p1_hbm, w0_hbm, w1_hbm, res_hbm,
          p0_v, p1_v, w0_v, w1_v,
          r0_a, r1_a, r0_b, r1_b, res_a,
          sem_a, sem_b):
        wid = lax.axis_index("s") * SC_NC + lax.axis_index("c")
        base = wid * TOK_PER_W
        pltpu.sync_copy(p0_hbm.at[pl.ds(base, TOK_PER_W)], p0_v)
        pltpu.sync_copy(p1_hbm.at[pl.ds(base, TOK_PER_W)], p1_v)
        pltpu.sync_copy(w0_hbm.at[pl.ds(base, TOK_PER_W)], w0_v)
        pltpu.sync_copy(w1_hbm.at[pl.ds(base, TOK_PER_W)], w1_v)

        def start(c):
            r0, r1 = (r0_a, r1_a) if c % 2 == 0 else (r0_b, r1_b)
            sem = sem_a if c % 2 == 0 else sem_b
            g0 = pltpu.async_copy(
                op_hbm.at[p0_v.at[pl.ds(c * C_CHUNK, C_CHUNK)]], r0, sem)
            g1 = pltpu.async_copy(
                op_hbm.at[p1_v.at[pl.ds(c * C_CHUNK, C_CHUNK)]], r1, sem)
            return g0, g1

        pending = start(0)
        for c in range(n_ch):
            r0, r1 = (r0_a, r1_a) if c % 2 == 0 else (r0_b, r1_b)
            res = res_a
            g0, g1 = pending
            if c + 1 < n_ch:
                pending = start(c + 1)
            g0.wait()
            g1.wait()

            def token_body(t, _):
                w0b = w0_v[c * C_CHUNK + t, pl.ds(0, 16)]
                w1b = w1_v[c * C_CHUNK + t, pl.ds(0, 16)]

                def vec_body(v, _):
                    for u in range(4):
                        sl = pl.ds(v * 64 + u * 16, 16)
                        res[t, sl] = w0b * r0[t, sl] + w1b * r1[t, sl]
                    return 0

                lax.fori_loop(0, n_grp, vec_body, 0)
                return 0

            lax.fori_loop(0, C_CHUNK, token_body, 0)
            pltpu.sync_copy(
                res, res_hbm.at[pl.ds(base + c * C_CHUNK, C_CHUNK)])

    return k(out_perm, ppos0, ppos1, ew0, ew1)


def kernel(x, expert_weights, expert_indices, w1, w2):
    in_shape = x.shape

    w1b, w2b = _cast_weights(w1, w2)

    ppos0_2d, ppos1_2d, binfo = _route(expert_indices)
    block_expert = binfo[0, :NB]
    nact = binfo[1, :1]

    def bgroup(p):
        return p.reshape(NTOK // G_CHUNK, G_CHUNK // 2, 2).transpose(
            0, 2, 1).reshape(NTOK // G_CHUNK, G_CHUNK)

    gathered = _sc_gather(x, bgroup(ppos0_2d), bgroup(ppos1_2d))
    out_perm = _grouped_mlp(block_expert, nact, gathered, w1b, w2b)

    ew = expert_weights.astype(jnp.float32)
    ew0 = jnp.broadcast_to(ew[:, 0:1], (NTOK, 16))
    ew1 = jnp.broadcast_to(ew[:, 1:2], (NTOK, 16))
    result = _sc_combine(
        out_perm, ppos0_2d.reshape(NTOK), ppos1_2d.reshape(NTOK), ew0, ew1)
    return result.reshape(in_shape)


# speedup vs baseline: 1.2639x; 1.0508x over previous
import functools

import jax
import jax.numpy as jnp
from jax import lax
from jax.experimental import pallas as pl
from jax.experimental.pallas import tpu as pltpu
from jax.experimental.pallas import tpu_sc as plsc

NUM_EXPERTS = 8
TOP_K = 2
D_MODEL = 1024
D_FF = 4096
NTOK = 4096
NE = NTOK * TOP_K
BLOCK = 256
NPAD = NE + NUM_EXPERTS * BLOCK
NB = NPAD // BLOCK
TOK_R, TOK_C = 32, 128

SC_NC = 2
SC_NW = 32
TOK_PER_W = NTOK // SC_NW
G_CHUNK = 32
C_CHUNK = 16


def _routing_kernel(idx0_ref, idx1_ref, ppos0_ref, ppos1_ref, binfo_ref):
    idx0 = idx0_ref[...]
    idx1 = idx1_ref[...]
    r = lax.broadcasted_iota(jnp.int32, (TOK_C, TOK_C), 0)
    c = lax.broadcasted_iota(jnp.int32, (TOK_C, TOK_C), 1)
    upper = (r < c).astype(jnp.float32)
    r2 = lax.broadcasted_iota(jnp.int32, (TOK_R, TOK_R), 0)
    c2 = lax.broadcasted_iota(jnp.int32, (TOK_R, TOK_R), 1)
    lower = (c2 < r2).astype(jnp.float32)
    ones = jnp.ones((TOK_C, TOK_C), jnp.float32)

    ppos0 = jnp.zeros((TOK_R, TOK_C), jnp.int32)
    ppos1 = jnp.zeros((TOK_R, TOK_C), jnp.int32)
    off = jnp.int32(0)
    ends = []
    for e in range(NUM_EXPERTS):
        m0 = (idx0 == e)
        m1 = (idx1 == e)
        m0f = m0.astype(jnp.float32)
        s = m0f + m1.astype(jnp.float32)
        within_row = jnp.dot(s, upper, preferred_element_type=jnp.float32)
        before_rows = jnp.dot(
            jnp.dot(lower, s, preferred_element_type=jnp.float32), ones,
            preferred_element_type=jnp.float32)
        pref = within_row + before_rows
        rank0 = pref.astype(jnp.int32)
        rank1 = (pref + m0f).astype(jnp.int32)
        cnt = jnp.sum(s).astype(jnp.int32)
        padded_cnt = ((cnt + BLOCK - 1) // BLOCK) * BLOCK
        ppos0 = ppos0 + m0.astype(jnp.int32) * (off + rank0)
        ppos1 = ppos1 + m1.astype(jnp.int32) * (off + rank1)
        off = off + padded_cnt
        ends.append(off)

    ppos0_ref[...] = ppos0
    ppos1_ref[...] = ppos1
    block_start = lax.broadcasted_iota(jnp.int32, (8, 128), 1) * BLOCK
    be = jnp.zeros((8, 128), jnp.int32)
    for e in range(NUM_EXPERTS):
        be = be + (block_start >= ends[e]).astype(jnp.int32)
    be = jnp.minimum(be, NUM_EXPERTS - 1)
    row = lax.broadcasted_iota(jnp.int32, (8, 128), 0)
    binfo_ref[...] = jnp.where(row == 0, be, off // BLOCK)


def _route(expert_indices):
    ei = expert_indices.astype(jnp.int32)
    idx0 = ei[:, 0].reshape(TOK_R, TOK_C)
    idx1 = ei[:, 1].reshape(TOK_R, TOK_C)
    return pl.pallas_call(
        _routing_kernel,
        out_shape=(
            jax.ShapeDtypeStruct((TOK_R, TOK_C), jnp.int32),
            jax.ShapeDtypeStruct((TOK_R, TOK_C), jnp.int32),
            jax.ShapeDtypeStruct((8, 128), jnp.int32),
        ),
    )(idx0, idx1)


def _gemm_kernel(be_ref, nact_ref, xg_ref, w1_ref, w2_ref, out_ref):
    del be_ref
    b = pl.program_id(0)

    @pl.when(b < nact_ref[0])
    def _():
        mid = jax.nn.gelu(
            jnp.dot(xg_ref[...].astype(jnp.bfloat16), w1_ref[0],
                    preferred_element_type=jnp.float32))
        out_ref[...] = jnp.dot(mid.astype(jnp.bfloat16), w2_ref[0],
                               preferred_element_type=jnp.float32)


def _grouped_mlp(block_expert, nact, gathered, w1b, w2b):
    grid_spec = pltpu.PrefetchScalarGridSpec(
        num_scalar_prefetch=2,
        grid=(NB,),
        in_specs=[
            pl.BlockSpec((BLOCK, D_MODEL), lambda b, be, na: (b, 0)),
            pl.BlockSpec((1, D_MODEL, D_FF), lambda b, be, na: (be[b], 0, 0)),
            pl.BlockSpec((1, D_FF, D_MODEL), lambda b, be, na: (be[b], 0, 0)),
        ],
        out_specs=pl.BlockSpec((BLOCK, D_MODEL), lambda b, be, na: (b, 0)),
    )
    return pl.pallas_call(
        _gemm_kernel,
        grid_spec=grid_spec,
        out_shape=jax.ShapeDtypeStruct((NPAD, D_MODEL), jnp.float32),
    )(block_expert, nact, gathered, w1b, w2b)


def _cast_kernel(w1_ref, w2_ref, o1_ref, o2_ref):
    o1_ref[...] = w1_ref[...].astype(jnp.bfloat16)
    o2_ref[...] = w2_ref[...].astype(jnp.bfloat16)


def _cast_weights(w1, w2):
    return pl.pallas_call(
        _cast_kernel,
        grid=(NUM_EXPERTS, 4),
        in_specs=[
            pl.BlockSpec((1, D_MODEL // 4, D_FF), lambda e, i: (e, i, 0)),
            pl.BlockSpec((1, D_FF // 4, D_MODEL), lambda e, i: (e, i, 0)),
        ],
        out_specs=[
            pl.BlockSpec((1, D_MODEL // 4, D_FF), lambda e, i: (e, i, 0)),
            pl.BlockSpec((1, D_FF // 4, D_MODEL), lambda e, i: (e, i, 0)),
        ],
        out_shape=[
            jax.ShapeDtypeStruct((NUM_EXPERTS, D_MODEL, D_FF), jnp.bfloat16),
            jax.ShapeDtypeStruct((NUM_EXPERTS, D_FF, D_MODEL), jnp.bfloat16),
        ],
    )(w1, w2)


def _sc_gather(x_flat, ppos0_2d, ppos1_2d):
    mesh = plsc.VectorSubcoreMesh(core_axis_name="c", subcore_axis_name="s")
    n_ch = TOK_PER_W // G_CHUNK
    n_buf = 3

    @functools.partial(
        pl.kernel, mesh=mesh,
        out_type=jax.ShapeDtypeStruct((NPAD, D_MODEL), jnp.float32),
        scratch_types=[
            pltpu.VMEM((n_ch, G_CHUNK), jnp.int32),
            pltpu.VMEM((n_ch, G_CHUNK), jnp.int32),
        ] + [pltpu.VMEM((G_CHUNK, D_MODEL), jnp.float32)] * n_buf + [
            pltpu.SemaphoreType.DMA,
            pltpu.SemaphoreType.DMA,
            pltpu.SemaphoreType.DMA,
        ],
    )
    def k(x_hbm, p0_hbm, p1_hbm, out_hbm, p0_v, p1_v,
          buf0, buf1, buf2, semg, sem0, sem1):
        bufs = [buf0, buf1, buf2]
        wid = lax.axis_index("s") * SC_NC + lax.axis_index("c")
        base = wid * TOK_PER_W
        pltpu.sync_copy(p0_hbm.at[pl.ds(wid * n_ch, n_ch)], p0_v)
        pltpu.sync_copy(p1_hbm.at[pl.ds(wid * n_ch, n_ch)], p1_v)

        half = G_CHUNK // 2

        def gstart(c):
            s0 = (base + c * G_CHUNK) // 2
            buf = bufs[c % n_buf]
            a = pltpu.async_copy(
                x_hbm.at[pl.ds(s0, half), 0], buf.at[pl.ds(0, half)], semg)
            b = pltpu.async_copy(
                x_hbm.at[pl.ds(s0, half), 1], buf.at[pl.ds(half, half)],
                semg)
            return (a, b)

        g, sc0, sc1 = {}, {}, {}
        g[0] = gstart(0)
        g[1] = gstart(1)
        for c in range(n_ch):
            g[c][0].wait()
            g[c][1].wait()
            n = c + 2
            if n < n_ch:
                if n - n_buf >= 0:
                    sc0[n - n_buf].wait()
                    sc1[n - n_buf].wait()
                g[n] = gstart(n)
            sc0[c] = pltpu.async_copy(
                bufs[c % n_buf], out_hbm.at[p0_v.at[c]], sem0)
            sc1[c] = pltpu.async_copy(
                bufs[c % n_buf], out_hbm.at[p1_v.at[c]], sem1)
        for c in range(max(0, n_ch - n_buf), n_ch):
            sc0[c].wait()
            sc1[c].wait()

    return k(x_flat, ppos0_2d, ppos1_2d)


def _sc_combine(out_perm, ppos0, ppos1, ew0, ew1, out_3d_shape):
    mesh = plsc.VectorSubcoreMesh(core_axis_name="c", subcore_axis_name="s")
    n_ch = TOK_PER_W // C_CHUNK
    n_grp = D_MODEL // 64

    @functools.partial(
        pl.kernel, mesh=mesh,
        out_type=jax.ShapeDtypeStruct(out_3d_shape, jnp.float32),
        scratch_types=[
            pltpu.VMEM((TOK_PER_W,), jnp.int32),
            pltpu.VMEM((TOK_PER_W,), jnp.int32),
            pltpu.VMEM((TOK_PER_W, 16), jnp.float32),
            pltpu.VMEM((TOK_PER_W, 16), jnp.float32),
            pltpu.VMEM((C_CHUNK, D_MODEL), jnp.float32),
            pltpu.VMEM((C_CHUNK, D_MODEL), jnp.float32),
            pltpu.VMEM((C_CHUNK, D_MODEL), jnp.float32),
            pltpu.VMEM((C_CHUNK, D_MODEL), jnp.float32),
            pltpu.VMEM((C_CHUNK, D_MODEL), jnp.float32),
            pltpu.SemaphoreType.DMA,
            pltpu.SemaphoreType.DMA,
        ],
    )
    def k(op_hbm, p0_hbm, p1_hbm, w0_hbm, w1_hbm, res_hbm,
          p0_v, p1_v, w0_v, w1_v,
          r0_a, r1_a, r0_b, r1_b, res_a,
          sem_a, sem_b):
        wid = lax.axis_index("s") * SC_NC + lax.axis_index("c")
        base = wid * TOK_PER_W
        pltpu.sync_copy(p0_hbm.at[pl.ds(base, TOK_PER_W)], p0_v)
        pltpu.sync_copy(p1_hbm.at[pl.ds(base, TOK_PER_W)], p1_v)
        pltpu.sync_copy(w0_hbm.at[pl.ds(base, TOK_PER_W)], w0_v)
        pltpu.sync_copy(w1_hbm.at[pl.ds(base, TOK_PER_W)], w1_v)

        def start(c):
            r0, r1 = (r0_a, r1_a) if c % 2 == 0 else (r0_b, r1_b)
            sem = sem_a if c % 2 == 0 else sem_b
            g0 = pltpu.async_copy(
                op_hbm.at[p0_v.at[pl.ds(c * C_CHUNK, C_CHUNK)]], r0, sem)
            g1 = pltpu.async_copy(
                op_hbm.at[p1_v.at[pl.ds(c * C_CHUNK, C_CHUNK)]], r1, sem)
            return g0, g1

        pending = start(0)
        for c in range(n_ch):
            r0, r1 = (r0_a, r1_a) if c % 2 == 0 else (r0_b, r1_b)
            res = res_a
            g0, g1 = pending
            if c + 1 < n_ch:
                pending = start(c + 1)
            g0.wait()
            g1.wait()

            def token_body(t, _):
                w0b = w0_v[c * C_CHUNK + t, pl.ds(0, 16)]
                w1b = w1_v[c * C_CHUNK + t, pl.ds(0, 16)]

                def vec_body(v, _):
                    for u in range(4):
                        sl = pl.ds(v * 64 + u * 16, 16)
                        res[t, sl] = w0b * r0[t, sl] + w1b * r1[t, sl]
                    return 0

                lax.fori_loop(0, n_grp, vec_body, 0)
                return 0

            lax.fori_loop(0, C_CHUNK, token_body, 0)
            half = C_CHUNK // 2
            s0 = (base + c * C_CHUNK) // 2
            pltpu.sync_copy(
                res.at[pl.ds(0, half)], res_hbm.at[pl.ds(s0, half), 0])
            pltpu.sync_copy(
                res.at[pl.ds(half, half)], res_hbm.at[pl.ds(s0, half), 1])

    return k(out_perm, ppos0, ppos1, ew0, ew1)


def kernel(x, expert_weights, expert_indices, w1, w2):
    in_shape = x.shape

    w1b, w2b = _cast_weights(w1, w2)

    ppos0_2d, ppos1_2d, binfo = _route(expert_indices)
    block_expert = binfo[0, :NB]
    nact = binfo[1, :1]

    def bgroup(p):
        return p.reshape(NTOK // G_CHUNK, G_CHUNK // 2, 2).transpose(
            0, 2, 1).reshape(NTOK // G_CHUNK, G_CHUNK)

    gathered = _sc_gather(x, bgroup(ppos0_2d), bgroup(ppos1_2d))
    out_perm = _grouped_mlp(block_expert, nact, gathered, w1b, w2b)

    def cgroup(v):
        return v.reshape(NTOK // C_CHUNK, C_CHUNK // 2, 2).transpose(
            0, 2, 1).reshape(NTOK)

    ew = expert_weights.astype(jnp.float32)
    ew0 = jnp.broadcast_to(cgroup(ew[:, 0])[:, None], (NTOK, 16))
    ew1 = jnp.broadcast_to(cgroup(ew[:, 1])[:, None], (NTOK, 16))
    return _sc_combine(
        out_perm, cgroup(ppos0_2d.reshape(NTOK)),
        cgroup(ppos1_2d.reshape(NTOK)), ew0, ew1, in_shape)
